# baseline (speedup 1.0000x reference)
import jax
import jax.numpy as jnp
from jax.experimental import pallas as pl
from jax.experimental.pallas import tpu as pltpu

_BLOCK = 4096


def _body(x_ref, wr_ref, wb_ref, o_ref):
    x = x_ref[...]
    e = wb_ref.shape[0]
    d = x_ref.shape[1]
    logits = jnp.dot(x, wr_ref[...], preferred_element_type=jnp.float32)
    values = jnp.argmax(logits, axis=-1)
    iota = jax.lax.broadcasted_iota(jnp.int32, (1, e), 1)
    onehot = (values[:, None] == iota).astype(jnp.float32)
    wb_tok = jnp.dot(onehot, wb_ref[...], preferred_element_type=jnp.float32)
    o_ref[...] = jnp.maximum(x * wb_tok[:, :d] + wb_tok[:, d:], 0.0)


def kernel(x, w_router, w_expert, b_expert):
    n, d = x.shape
    e = w_router.shape[1]
    block = min(_BLOCK, n)
    wb = jnp.concatenate([w_expert, b_expert], axis=1)
    return pl.pallas_call(
        _body,
        grid=(n // block,),
        in_specs=[
            pl.BlockSpec((block, d), lambda i: (i, 0)),
            pl.BlockSpec((d, e), lambda i: (0, 0)),
            pl.BlockSpec((e, 2 * d), lambda i: (0, 0)),
        ],
        out_specs=pl.BlockSpec((block, d), lambda i: (i, 0)),
        out_shape=jax.ShapeDtypeStruct((n, d), jnp.float32),
        compiler_params=pltpu.CompilerParams(
            dimension_semantics=("parallel",),
        ),
    )(x, w_router, wb)


# speedup vs baseline: 1.0963x; 1.0963x over previous
import jax
import jax.numpy as jnp
from jax.experimental import pallas as pl
from jax.experimental.pallas import tpu as pltpu

_BLOCK = 4096
_SUB = 2


def _body(x_ref, wr_ref, we_ref, be_ref, o_ref):
    e = we_ref.shape[0]
    block = x_ref.shape[0]
    sub = block // _SUB
    wr = wr_ref[...]
    we = we_ref[...]
    be = be_ref[...]
    iota = jax.lax.broadcasted_iota(jnp.int32, (1, e), 1)
    for h in range(_SUB):
        x = x_ref[h * sub : (h + 1) * sub, :]
        logits = jnp.dot(x, wr, preferred_element_type=jnp.float32)
        values = jnp.argmax(logits, axis=-1)
        onehot = (values[:, None] == iota).astype(jnp.float32)
        w_tok = jnp.dot(onehot, we, preferred_element_type=jnp.float32)
        b_tok = jnp.dot(onehot, be, preferred_element_type=jnp.float32)
        o_ref[h * sub : (h + 1) * sub, :] = jnp.maximum(x * w_tok + b_tok, 0.0)


def kernel(x, w_router, w_expert, b_expert):
    n, d = x.shape
    e = w_router.shape[1]
    block = min(_BLOCK, n)
    return pl.pallas_call(
        _body,
        grid=(n // block,),
        in_specs=[
            pl.BlockSpec((block, d), lambda i: (i, 0)),
            pl.BlockSpec((d, e), lambda i: (0, 0)),
            pl.BlockSpec((e, d), lambda i: (0, 0)),
            pl.BlockSpec((e, d), lambda i: (0, 0)),
        ],
        out_specs=pl.BlockSpec((block, d), lambda i: (i, 0)),
        out_shape=jax.ShapeDtypeStruct((n, d), jnp.float32),
        compiler_params=pltpu.CompilerParams(
            dimension_semantics=("parallel",),
        ),
    )(x, w_router, w_expert, b_expert)


# speedup vs baseline: 1.1729x; 1.0698x over previous
import jax
import jax.numpy as jnp
from jax.experimental import pallas as pl
from jax.experimental.pallas import tpu as pltpu

_BLOCK = 4096
_SUB = 2


def _body(x_ref, wr_ref, we_ref, be_ref, o_ref):
    e = we_ref.shape[0]
    block = x_ref.shape[0]
    sub = block // _SUB
    wr = wr_ref[...]
    we = we_ref[...]
    be = be_ref[...]
    iota = jax.lax.broadcasted_iota(jnp.int32, (1, e), 1)
    for h in range(_SUB):
        x = x_ref[h * sub : (h + 1) * sub, :]
        logits = jnp.dot(x, wr, preferred_element_type=jnp.float32)
        values = jnp.argmax(logits, axis=-1)
        vb = jnp.broadcast_to(values[:, None], x.shape).astype(jnp.int32)
        w_tok = jnp.take_along_axis(we, vb, axis=0)
        b_tok = jnp.take_along_axis(be, vb, axis=0)
        o_ref[h * sub : (h + 1) * sub, :] = jnp.maximum(x * w_tok + b_tok, 0.0)


def kernel(x, w_router, w_expert, b_expert):
    n, d = x.shape
    e = w_router.shape[1]
    block = min(_BLOCK, n)
    return pl.pallas_call(
        _body,
        grid=(n // block,),
        in_specs=[
            pl.BlockSpec((block, d), lambda i: (i, 0)),
            pl.BlockSpec((d, e), lambda i: (0, 0)),
            pl.BlockSpec((e, d), lambda i: (0, 0)),
            pl.BlockSpec((e, d), lambda i: (0, 0)),
        ],
        out_specs=pl.BlockSpec((block, d), lambda i: (i, 0)),
        out_shape=jax.ShapeDtypeStruct((n, d), jnp.float32),
        compiler_params=pltpu.CompilerParams(
            dimension_semantics=("parallel",),
        ),
    )(x, w_router, w_expert, b_expert)


# speedup vs baseline: 1.1885x; 1.0133x over previous
import jax
import jax.numpy as jnp
from jax.experimental import pallas as pl
from jax.experimental.pallas import tpu as pltpu

_BLOCK = 4096
_SUB = 1


def _body(x_ref, wr_ref, we_ref, be_ref, o_ref):
    e = we_ref.shape[0]
    block = x_ref.shape[0]
    sub = block // _SUB
    wr = wr_ref[...]
    we = we_ref[...]
    be = be_ref[...]
    iota = jax.lax.broadcasted_iota(jnp.int32, (1, e), 1)
    for h in range(_SUB):
        x = x_ref[h * sub : (h + 1) * sub, :]
        logits = jnp.dot(x, wr, preferred_element_type=jnp.float32)
        values = jnp.argmax(logits, axis=-1)
        vb = jnp.broadcast_to(values[:, None], x.shape).astype(jnp.int32)
        w_tok = jnp.take_along_axis(we, vb, axis=0)
        b_tok = jnp.take_along_axis(be, vb, axis=0)
        o_ref[h * sub : (h + 1) * sub, :] = jnp.maximum(x * w_tok + b_tok, 0.0)


def kernel(x, w_router, w_expert, b_expert):
    n, d = x.shape
    e = w_router.shape[1]
    block = min(_BLOCK, n)
    return pl.pallas_call(
        _body,
        grid=(n // block,),
        in_specs=[
            pl.BlockSpec((block, d), lambda i: (i, 0)),
            pl.BlockSpec((d, e), lambda i: (0, 0)),
            pl.BlockSpec((e, d), lambda i: (0, 0)),
            pl.BlockSpec((e, d), lambda i: (0, 0)),
        ],
        out_specs=pl.BlockSpec((block, d), lambda i: (i, 0)),
        out_shape=jax.ShapeDtypeStruct((n, d), jnp.float32),
        compiler_params=pltpu.CompilerParams(
            dimension_semantics=("parallel",),
        ),
    )(x, w_router, w_expert, b_expert)
